# baseline (speedup 1.0000x reference)
import functools

import jax
import jax.numpy as jnp
from jax import lax
from jax.experimental import pallas as pl
from jax.experimental.pallas import tpu as pltpu
from jax.experimental.pallas import tpu_sc as plsc

F32 = jnp.float32
_N = 50000
_E = 800000
_G = 128
_C = 64
_HC = 32
_NP = 50176
_BLK = 3584
_NB = _NP // _BLK
_NTILE = 16
_RPT = _NP // _NTILE
_EPT = _E // _NTILE
_EK = 2000
_ECH = _EPT // _EK
_EKA = 400
_ECHA = _EPT // _EKA
_KP = _RPT // 2
_EPS = 1e-5
_PREC = lax.Precision.HIGHEST


@functools.cache
def _sc_mesh():
    return plsc.VectorSubcoreMesh(core_axis_name="c", subcore_axis_name="s",
                                  num_cores=2, num_subcores=_NTILE)


def _make_agg_body(with_deg):
    def body(*refs):
        it = iter(refs)
        r_lo, r_hi, src, tgt = (next(it) for _ in range(4))
        af_lo, af_hi, ab_lo, ab_hi = (next(it) for _ in range(4))
        if with_deg:
            dgi, dgo = next(it), next(it)
        (ig0, ig1, it00, it01, it10, it11, rows0, rows1, acc, zidx, zrows,
         sig0, sig1, sit0, sit1, sg0, sg1, ss0, ss1) = it

        c = lax.axis_index("c")
        s = lax.axis_index("s")
        igs, rws = (ig0, ig1), (rows0, rows1)
        its_ = ((it00, it01), (it10, it11))
        sigs, sits, sgs, sss = (sig0, sig1), (sit0, sit1), (sg0, sg1), (ss0, ss1)
        zidx[...] = lax.iota(jnp.int32, 16)
        for _zi in range(16):
            zrows[_zi, 0:16] = jnp.zeros((16,), F32)
            zrows[_zi, 16:32] = jnp.zeros((16,), F32)
        ebase = s * _EPT

        def issue_it(k, p, g, earr):
            off = ebase + k * _EKA
            pltpu.async_copy(earr.at[pl.ds(off, _EKA)], its_[p][g], sits[p])

        def wait_it(p, g, earr):
            pltpu.make_async_copy(earr.at[pl.ds(ebase, _EKA)], its_[p][g],
                                  sits[p]).wait()

        def run_deg(earr, dout):
            def fill(i, carry):
                rows0[i, 0:16] = jnp.ones((16,), F32)
                rows0[i, 16:32] = jnp.ones((16,), F32)
                return carry

            lax.fori_loop(0, _EKA, fill, 0)

            def init(i, carry):
                pltpu.sync_copy(rows0,
                                acc.at[pl.ds(s * _RPT + i * _EKA, _EKA)])
                return carry

            lax.fori_loop(0, _RPT // _EKA, init, 0)
            pltpu.sync_copy(rows0.at[pl.ds(0, _RPT % _EKA)],
                            acc.at[pl.ds(s * _RPT + _RPT - _RPT % _EKA,
                                         _RPT % _EKA)])
            plsc.subcore_barrier()

            def dchunk(k, carry):
                pltpu.sync_copy(earr.at[pl.ds(ebase + k * _EKA, _EKA)], it00)
                pltpu.sync_copy(rows0, acc.at[it00], add=True)
                return carry

            lax.fori_loop(0, _ECHA, dchunk, 0)
            pltpu.sync_copy(zrows, acc.at[zidx], add=True)
            plsc.subcore_barrier()
            pltpu.sync_copy(acc.at[pl.ds(s * _RPT, _RPT)],
                            dout.at[pl.ds(s * _RPT, _RPT)])
            plsc.subcore_barrier()

        def direction(rh, garr, tarr, out):
            pltpu.sync_copy(rh.at[pl.ds(s * _RPT, _RPT)],
                            acc.at[pl.ds(s * _RPT, _RPT)])
            plsc.subcore_barrier()

            def issue_idx(k, p, g):
                off = ebase + k * _EKA
                pltpu.async_copy(garr.at[pl.ds(off, _EKA)], igs[p], sigs[p])
                issue_it(k, p, g, tarr)

            def wait_idx(p, g):
                pltpu.make_async_copy(garr.at[pl.ds(ebase, _EKA)], igs[p],
                                      sigs[p]).wait()
                wait_it(p, g, tarr)

            def issue_gather(p):
                pltpu.async_copy(rh.at[igs[p]], rws[p], sgs[p])

            def wait_gather(p):
                pltpu.make_async_copy(rh.at[igs[p]], rws[p], sgs[p]).wait()

            def issue_scatter(p, g):
                pltpu.async_copy(rws[p], acc.at[its_[p][g]], sss[p], add=True)

            def wait_scatter(p, g):
                pltpu.make_async_copy(rws[p], acc.at[its_[p][g]],
                                      sss[p]).wait()

            def process(p, g, gn, gp, k):
                q = 1 - p
                wait_gather(p)

                @pl.when(k + 1 < _ECHA)
                def _():
                    wait_idx(q, gn)

                    @pl.when(k >= 1)
                    def _():
                        wait_scatter(q, gp)

                    issue_gather(q)

                issue_scatter(p, g)

                @pl.when(k + 2 < _ECHA)
                def _():
                    issue_idx(k + 2, p, 1 - g)

            issue_idx(0, 0, 0)
            issue_idx(1, 1, 0)
            wait_idx(0, 0)
            issue_gather(0)

            def lbody(k2, carry):
                k = 4 * k2
                process(0, 0, 0, 1, k)
                process(1, 0, 1, 0, k + 1)
                process(0, 1, 1, 0, k + 2)
                process(1, 1, 0, 1, k + 3)
                return carry

            lax.fori_loop(0, _ECHA // 4, lbody, 0)
            process(0, 0, 0, 1, _ECHA - 1)
            wait_scatter(1, 1)
            wait_scatter(0, 0)
            pltpu.sync_copy(zrows, acc.at[zidx], add=True)
            plsc.subcore_barrier()
            pltpu.sync_copy(acc.at[pl.ds(s * _RPT, _RPT)],
                            out.at[pl.ds(s * _RPT, _RPT)])
            plsc.subcore_barrier()

        @pl.when(c == 0)
        def _():
            if with_deg:
                run_deg(tgt, dgi)
            direction(r_lo, src, tgt, af_lo)
            direction(r_lo, tgt, src, ab_lo)

        @pl.when(c == 1)
        def _():
            if with_deg:
                run_deg(src, dgo)
            direction(r_hi, src, tgt, af_hi)
            direction(r_hi, tgt, src, ab_hi)

    return body


@functools.cache
def _sc_agg(with_deg):
    n_out = 6 if with_deg else 4
    return pl.kernel(
        _make_agg_body(with_deg),
        out_type=[jax.ShapeDtypeStruct((_NP, _HC), F32)] * n_out,
        mesh=_sc_mesh(),
        compiler_params=pltpu.CompilerParams(use_tc_tiling_on_sc=False),
        scratch_types=[
            pltpu.VMEM((_EKA,), jnp.int32),
            pltpu.VMEM((_EKA,), jnp.int32),
            pltpu.VMEM((_EKA,), jnp.int32),
            pltpu.VMEM((_EKA,), jnp.int32),
            pltpu.VMEM((_EKA,), jnp.int32),
            pltpu.VMEM((_EKA,), jnp.int32),
            pltpu.VMEM((_EKA, _HC), F32),
            pltpu.VMEM((_EKA, _HC), F32),
            pltpu.VMEM_SHARED((_NP, _HC), F32),
            pltpu.VMEM((16,), jnp.int32),
            pltpu.VMEM((16, _HC), F32),
            pltpu.SemaphoreType.DMA,
            pltpu.SemaphoreType.DMA,
            pltpu.SemaphoreType.DMA,
            pltpu.SemaphoreType.DMA,
            pltpu.SemaphoreType.DMA,
            pltpu.SemaphoreType.DMA,
            pltpu.SemaphoreType.DMA,
            pltpu.SemaphoreType.DMA,
        ],
    )


def _sc_pool_body(h_lo, h_hi, batchp, p_lo, p_hi, idxbuf, rows, acc):
    c = lax.axis_index("c")
    s = lax.axis_index("s")

    @pl.when(s == 0)
    def _init():
        def fill(i, carry):
            rows[i, 0:16] = jnp.zeros((16,), F32)
            rows[i, 16:32] = jnp.zeros((16,), F32)
            return carry

        lax.fori_loop(0, _G, fill, 0)
        pltpu.sync_copy(rows.at[pl.ds(0, _G)], acc)

    plsc.subcore_barrier()

    def run(harr, out):
        def chunk(k, carry):
            roff = s * _RPT + k * _KP
            pltpu.sync_copy(harr.at[pl.ds(roff, _KP)], rows)
            pltpu.sync_copy(batchp.at[pl.ds(roff, _KP)], idxbuf)
            pltpu.sync_copy(rows, acc.at[idxbuf], add=True)
            return carry

        lax.fori_loop(0, 2, chunk, 0)
        plsc.subcore_barrier()

        @pl.when(s == 0)
        def _out():
            pltpu.sync_copy(acc, out)

    @pl.when(c == 0)
    def _():
        run(h_lo, p_lo)

    @pl.when(c == 1)
    def _():
        run(h_hi, p_hi)


@functools.cache
def _sc_pool():
    return pl.kernel(
        _sc_pool_body,
        out_type=[jax.ShapeDtypeStruct((_G, _HC), F32)] * 2,
        mesh=_sc_mesh(),
        compiler_params=pltpu.CompilerParams(use_tc_tiling_on_sc=False),
        scratch_types=[
            pltpu.VMEM((_KP,), jnp.int32),
            pltpu.VMEM((_KP, _HC), F32),
            pltpu.VMEM_SHARED((_G, _HC), F32),
        ],
    )


def _tc_embed_body(xf, emb, h_lo, h_hi):
    i = pl.program_id(0)
    xv = xf[...]
    e = emb[...]
    h = jnp.zeros((_BLK, _C), F32)
    for k in range(7):
        h = h + jnp.where(xv == float(k), 1.0, 0.0) * e[k][None, :]
    rowid = i * _BLK + lax.broadcasted_iota(jnp.int32, (_BLK, 1), 0)
    h = jnp.where(rowid < _N, h, 0.0)
    h_lo[...] = h[:, :_HC]
    h_hi[...] = h[:, _HC:]


def _tc_embed(xf, emb):
    return pl.pallas_call(
        _tc_embed_body,
        grid=(_NB,),
        in_specs=[
            pl.BlockSpec((_BLK, 1), lambda i: (i, 0)),
            pl.BlockSpec((7, _C), lambda i: (0, 0)),
        ],
        out_specs=[
            pl.BlockSpec((_BLK, _HC), lambda i: (i, 0)),
            pl.BlockSpec((_BLK, _HC), lambda i: (i, 0)),
        ],
        out_shape=[jax.ShapeDtypeStruct((_NP, _HC), F32),
                   jax.ShapeDtypeStruct((_NP, _HC), F32)],
    )(xf, emb)


def _tc_stage(aggs, degi, dego, w0, w1, sums, save, first, resid, last):
    def body(*refs):
        it = iter(refs)
        aggf_lo, aggf_hi, aggb_lo, aggb_hi, degi_r, dego_r, w0_r, w1_r = (
            next(it) for _ in range(8))
        sum_r = sumsq_r = save_r = None
        if not first:
            sum_r, sumsq_r = next(it), next(it)
        if resid:
            save_r = next(it)
        if last:
            h_lo, h_hi = next(it), next(it)
        else:
            r_lo, r_hi, sum_o, sumsq_o = (next(it) for _ in range(4))
            save_o = next(it) if (first or resid) else None

        i = pl.program_id(0)
        aggf = jnp.concatenate([aggf_lo[...], aggf_hi[...]], axis=1)
        aggb = jnp.concatenate([aggb_lo[...], aggb_hi[...]], axis=1)
        norm = 1.0 / degi_r[:, 0:1]
        normt = 1.0 / dego_r[:, 0:1]
        w0v, w1v = w0_r[...], w1_r[...]
        if not first:
            m = jnp.sum(sum_r[...], axis=(0, 1)) * (1.0 / _N)
            var = jnp.sum(sumsq_r[...], axis=(0, 1)) * (1.0 / _N) - m * m
            inv = lax.rsqrt(var + _EPS)
            aggf = aggf * inv[None, :]
            aggb = aggb * inv[None, :]
        h = (jnp.dot(norm * aggf, w0v, precision=_PREC) +
             jnp.dot(normt * aggb, w1v, precision=_PREC))
        if not first:
            bias = jnp.dot((m * inv)[None, :], w0v + w1v, precision=_PREC)
            h = h - bias
        if resid:
            h = h + save_r[...]
        rowid = i * _BLK + lax.broadcasted_iota(jnp.int32, (_BLK, 1), 0)
        h = jnp.where(rowid < _N, h, 0.0)
        if last:
            h_lo[...] = h[:, :_HC]
            h_hi[...] = h[:, _HC:]
        else:
            if save_o is not None:
                save_o[...] = h
            r = jnp.maximum(h, 0.0)
            r_lo[...] = r[:, :_HC]
            r_hi[...] = r[:, _HC:]
            sum_o[...] = jnp.sum(r, axis=0, keepdims=True)[None]
            sumsq_o[...] = jnp.sum(r * r, axis=0, keepdims=True)[None]

    half = pl.BlockSpec((_BLK, _HC), lambda i: (i, 0))
    in_specs = [half, half, half, half,
                pl.BlockSpec((_BLK, _HC), lambda i: (i, 0)),
                pl.BlockSpec((_BLK, _HC), lambda i: (i, 0)),
                pl.BlockSpec((_C, _C), lambda i: (0, 0)),
                pl.BlockSpec((_C, _C), lambda i: (0, 0))]
    args = list(aggs) + [degi, dego, w0, w1]
    if not first:
        in_specs += [pl.BlockSpec((_NB, 1, _C), lambda i: (0, 0, 0))] * 2
        args += [sums[0], sums[1]]
    if resid:
        in_specs.append(pl.BlockSpec((_BLK, _C), lambda i: (i, 0)))
        args.append(save)
    if last:
        out_specs = [half, half]
        out_shape = [jax.ShapeDtypeStruct((_NP, _HC), F32)] * 2
    else:
        out_specs = [half, half,
                     pl.BlockSpec((1, 1, _C), lambda i: (i, 0, 0)),
                     pl.BlockSpec((1, 1, _C), lambda i: (i, 0, 0))]
        out_shape = [jax.ShapeDtypeStruct((_NP, _HC), F32),
                     jax.ShapeDtypeStruct((_NP, _HC), F32),
                     jax.ShapeDtypeStruct((_NB, 1, _C), F32),
                     jax.ShapeDtypeStruct((_NB, 1, _C), F32)]
        if first or resid:
            out_specs.append(pl.BlockSpec((_BLK, _C), lambda i: (i, 0)))
            out_shape.append(jax.ShapeDtypeStruct((_NP, _C), F32))
    return pl.pallas_call(
        body, grid=(_NB,), in_specs=in_specs, out_specs=out_specs,
        out_shape=out_shape,
    )(*args)


def _tc_head_body(p_lo, p_hi, cnt, f1w, f1b, f2w, f2b, f3w, f3b, out):
    p = jnp.concatenate([p_lo[...], p_hi[...]], axis=1) / cnt[...]
    g = jnp.maximum(jnp.dot(p, f1w[...], precision=_PREC) + f1b[...], 0.0)
    g = jnp.maximum(jnp.dot(g, f2w[...], precision=_PREC) + f2b[...], 0.0)
    out[...] = jnp.dot(g, f3w[...], precision=_PREC) + f3b[...]


def _tc_head(p_lo, p_hi, cnt, f1w, f1b, f2w, f2b, f3w, f3b):
    return pl.pallas_call(
        _tc_head_body,
        out_shape=jax.ShapeDtypeStruct((_G, 1), F32),
    )(p_lo, p_hi, cnt, f1w, f1b, f2w, f2b, f3w, f3b)


def kernel(x, sources, targets, batch, counts, total, emb, conv_w, res_w,
           fc1_w, fc1_b, fc2_w, fc2_b, fc3_w, fc3_b):
    del total
    pad = _NP - _N
    xf = jnp.pad(x, (0, pad)).astype(F32)[:, None]
    sources = sources.astype(jnp.int32)
    targets = targets.astype(jnp.int32)
    batchp = jnp.pad(batch.astype(jnp.int32), (0, pad))

    r_lo, r_hi = _tc_embed(xf, emb)
    degi = dego = None

    stage_w = [(conv_w[0], conv_w[1])]
    for i in range(4):
        for j in range(2):
            stage_w.append((res_w[i, j, 0], res_w[i, j, 1]))

    save = None
    sums = None
    out_final = None
    for k in range(9):
        outs_sc = _sc_agg(k == 0)(r_lo, r_hi, sources, targets)
        aggs = outs_sc[:4]
        if k == 0:
            degi, dego = outs_sc[4], outs_sc[5]
        first = k == 0
        resid = k in (2, 4, 6, 8)
        last = k == 8
        w0, w1 = stage_w[k]
        outs = _tc_stage(aggs, degi, dego, w0, w1, sums, save,
                         first, resid, last)
        if last:
            out_final = outs
        else:
            r_lo, r_hi, s_p, ss_p = outs[:4]
            if first or resid:
                save = outs[4]
            sums = (s_p, ss_p)

    p_lo, p_hi = _sc_pool()(out_final[0], out_final[1], batchp)
    out2 = _tc_head(p_lo, p_hi, counts[:, None], fc1_w, fc1_b[None, :],
                    fc2_w, fc2_b[None, :], fc3_w, fc3_b[None, :])
    return jnp.squeeze(out2, axis=-1)
